# initial kernel scaffold (unmeasured)
import jax
import jax.numpy as jnp
from jax import lax
from jax.experimental import pallas as pl
from jax.experimental.pallas import tpu as pltpu


def _exchange(x, d2):

    def body(x_ref, d_ref, px_ref, pd_ref, sx, sd, rx, rd):
        my_x = lax.axis_index("x")
        my_y = lax.axis_index("y")
        my_z = lax.axis_index("z")
        peer = (my_x, 1 - my_y, my_z)

        barrier = pltpu.get_barrier_semaphore()
        pl.semaphore_signal(
            barrier, inc=1, device_id=peer, device_id_type=pl.DeviceIdType.MESH
        )
        pl.semaphore_wait(barrier, 1)

        rdma_x = pltpu.make_async_remote_copy(
            src_ref=x_ref,
            dst_ref=px_ref,
            send_sem=sx,
            recv_sem=rx,
            device_id=peer,
            device_id_type=pl.DeviceIdType.MESH,
        )
        rdma_d = pltpu.make_async_remote_copy(
            src_ref=d_ref,
            dst_ref=pd_ref,
            send_sem=sd,
            recv_sem=rd,
            device_id=peer,
            device_id_type=pl.DeviceIdType.MESH,
        )
        rdma_x.start()
        rdma_d.start()
        rdma_x.wait()
        rdma_d.wait()

    return pl.pallas_call(
        body,
        out_shape=(
            jax.ShapeDtypeStruct(x.shape, x.dtype),
            jax.ShapeDtypeStruct(d2.shape, d2.dtype),
        ),
        in_specs=[
            pl.BlockSpec(memory_space=pltpu.VMEM),
            pl.BlockSpec(memory_space=pltpu.VMEM),
        ],
        out_specs=(
            pl.BlockSpec(memory_space=pltpu.VMEM),
            pl.BlockSpec(memory_space=pltpu.VMEM),
        ),
        scratch_shapes=[
            pltpu.SemaphoreType.DMA,
            pltpu.SemaphoreType.DMA,
            pltpu.SemaphoreType.DMA,
            pltpu.SemaphoreType.DMA,
        ],
        compiler_params=pltpu.CompilerParams(collective_id=0),
    )(x, d2)


def kernel(x, dest):
    t, _ = x.shape
    my_y = lax.axis_index("y")

    d2 = dest.reshape(8, -1)
    peer_x, peer_d2 = _exchange(x, d2)
    peer_dest = peer_d2.reshape(t)

    big_x = jnp.concatenate([x, peer_x], axis=0)
    big_d = jnp.concatenate([dest, peer_dest])
    origin = jnp.concatenate(
        [
            jnp.full((t,), my_y, jnp.int32),
            jnp.full((t,), 1 - my_y, jnp.int32),
        ]
    )
    key = big_d * 2 + origin
    order = jnp.argsort(key)
    sorted_x = jnp.take(big_x, order, axis=0)
    return lax.dynamic_slice_in_dim(sorted_x, my_y * t, t, axis=0)


# baseline (device time: 23374 ns/iter reference)
import jax
import jax.numpy as jnp
from jax import lax
from jax.experimental import pallas as pl
from jax.experimental.pallas import tpu as pltpu

T = 1024
D = 512
RS = 8
NCH = 2 * T // 128


def _a2a_kernel(x, d2):
    def body(x_ref, d_ref, out_ref, comm_x, comm_d, sems):
        my_x = lax.axis_index("x")
        my_y = lax.axis_index("y")
        my_z = lax.axis_index("z")
        peer = (my_x, 1 - my_y, my_z)

        comm_x[my_y] = x_ref[...].astype(jnp.bfloat16)
        comm_d[my_y] = d_ref[...]

        barrier = pltpu.get_barrier_semaphore()
        pl.semaphore_signal(
            barrier, inc=1, device_id=peer, device_id_type=pl.DeviceIdType.MESH
        )
        pl.semaphore_wait(barrier, 1)

        rdma_x = pltpu.make_async_remote_copy(
            src_ref=comm_x.at[my_y],
            dst_ref=comm_x.at[my_y],
            send_sem=sems.at[0],
            recv_sem=sems.at[1],
            device_id=peer,
            device_id_type=pl.DeviceIdType.MESH,
        )
        rdma_d = pltpu.make_async_remote_copy(
            src_ref=comm_d.at[my_y],
            dst_ref=comm_d.at[my_y],
            send_sem=sems.at[2],
            recv_sem=sems.at[3],
            device_id=peer,
            device_id_type=pl.DeviceIdType.MESH,
        )
        rdma_x.start()
        rdma_d.start()

        i0 = lax.broadcasted_iota(jnp.int32, (128, 128), 0)
        i1 = lax.broadcasted_iota(jnp.int32, (128, 128), 1)
        upper = (i0 <= i1).astype(jnp.float32)
        s0 = lax.broadcasted_iota(jnp.int32, (NCH, NCH), 0)
        s1 = lax.broadcasted_iota(jnp.int32, (NCH, NCH), 1)
        strict = (s1 < s0).astype(jnp.float32)
        jio = lax.broadcasted_iota(jnp.int32, (T, 128), 0)

        rdma_x.wait()
        rdma_d.wait()

        dall = comm_d[...].reshape(NCH, 128)
        m = (dall == my_y).astype(jnp.float32)
        rowpre = jnp.dot(m, upper, preferred_element_type=jnp.float32)
        totals = rowpre[:, 127:128]
        rowoff = jnp.dot(strict, totals, preferred_element_type=jnp.float32)
        pos = (rowoff + rowpre - m).astype(jnp.int32)

        acc = jnp.zeros((T, D), jnp.float32)
        for r in range(NCH):
            sel = (pos[r : r + 1, :] == jio) & (dall[r : r + 1, :] == my_y)
            p_r = sel.astype(jnp.bfloat16)
            chunk = comm_x[r // RS, (r % RS) * 128 : (r % RS + 1) * 128, :]
            acc = acc + jnp.dot(p_r, chunk, preferred_element_type=jnp.float32)
        out_ref[...] = acc

    return pl.pallas_call(
        body,
        out_shape=jax.ShapeDtypeStruct((T, D), jnp.float32),
        in_specs=[
            pl.BlockSpec(memory_space=pltpu.VMEM),
            pl.BlockSpec(memory_space=pltpu.VMEM),
        ],
        out_specs=pl.BlockSpec(memory_space=pltpu.VMEM),
        scratch_shapes=[
            pltpu.VMEM((2, T, D), jnp.bfloat16),
            pltpu.VMEM((2, RS, 128), jnp.int32),
            pltpu.SemaphoreType.DMA((4,)),
        ],
        compiler_params=pltpu.CompilerParams(collective_id=0),
    )(x, d2)


def kernel(x, dest):
    return _a2a_kernel(x, dest.reshape(RS, 128))


# device time: 19182 ns/iter; 1.2185x vs baseline; 1.2185x over previous
import jax
import jax.numpy as jnp
from jax import lax
from jax.experimental import pallas as pl
from jax.experimental.pallas import tpu as pltpu

T = 1024
D = 512
RS = 8
NCH = 2 * T // 128
NXC = 4
XC = T // NXC
MC = XC // 128


def _a2a_kernel(x, d2):
    def body(x_ref, d_ref, out_ref, comm_x, comm_d, sems):
        my_x = lax.axis_index("x")
        my_y = lax.axis_index("y")
        my_z = lax.axis_index("z")
        peer = (my_x, 1 - my_y, my_z)

        comm_d[my_y] = d_ref[...]
        comm_x[my_y] = x_ref[...].astype(jnp.bfloat16)

        barrier = pltpu.get_barrier_semaphore()
        pl.semaphore_signal(
            barrier, inc=1, device_id=peer, device_id_type=pl.DeviceIdType.MESH
        )
        pl.semaphore_wait(barrier, 1)

        rdma_d = pltpu.make_async_remote_copy(
            src_ref=comm_d.at[my_y],
            dst_ref=comm_d.at[my_y],
            send_sem=sems.at[0],
            recv_sem=sems.at[1],
            device_id=peer,
            device_id_type=pl.DeviceIdType.MESH,
        )
        rdma_d.start()
        rdma_x = []
        for k in range(NXC):
            r = pltpu.make_async_remote_copy(
                src_ref=comm_x.at[my_y, pl.ds(k * XC, XC), :],
                dst_ref=comm_x.at[my_y, pl.ds(k * XC, XC), :],
                send_sem=sems.at[2 + 2 * k],
                recv_sem=sems.at[3 + 2 * k],
                device_id=peer,
                device_id_type=pl.DeviceIdType.MESH,
            )
            r.start()
            rdma_x.append(r)

        i0 = lax.broadcasted_iota(jnp.int32, (128, 128), 0)
        i1 = lax.broadcasted_iota(jnp.int32, (128, 128), 1)
        upper = (i0 <= i1).astype(jnp.float32)
        s0 = lax.broadcasted_iota(jnp.int32, (NCH, NCH), 0)
        s1 = lax.broadcasted_iota(jnp.int32, (NCH, NCH), 1)
        strict = (s1 < s0).astype(jnp.float32)
        jio = lax.broadcasted_iota(jnp.int32, (T, 128), 0)

        rdma_d.wait()

        dall = comm_d[...].reshape(NCH, 128)
        m = (dall == my_y).astype(jnp.float32)
        rowpre = jnp.dot(m, upper, preferred_element_type=jnp.float32)
        totals = rowpre[:, 127:128]
        rowoff = jnp.dot(strict, totals, preferred_element_type=jnp.float32)
        pos = (rowoff + rowpre - m).astype(jnp.int32)

        im0 = my_y == 0
        pos_loc = jnp.where(im0, pos[:RS], pos[RS:])
        d_loc = jnp.where(im0, dall[:RS], dall[RS:])
        pos_rem = jnp.where(im0, pos[RS:], pos[:RS])
        d_rem = jnp.where(im0, dall[RS:], dall[:RS])

        def chunk_mm(acc, xslab, p8, d8, rr):
            sel = (p8[rr : rr + 1, :] == jio) & (d8[rr : rr + 1, :] == my_y)
            p_r = sel.astype(jnp.bfloat16)
            chunk = xslab[rr * 128 : (rr + 1) * 128, :]
            return acc + jnp.dot(p_r, chunk, preferred_element_type=jnp.float32)

        acc = jnp.zeros((T, D), jnp.float32)
        x_loc = comm_x[my_y]
        for rr in range(RS):
            acc = chunk_mm(acc, x_loc, pos_loc, d_loc, rr)

        for k in range(NXC):
            rdma_x[k].wait_recv()
            xslab = comm_x[1 - my_y, pl.ds(k * XC, XC), :]
            for h in range(MC):
                rr = k * MC + h
                sel = (pos_rem[rr : rr + 1, :] == jio) & (
                    d_rem[rr : rr + 1, :] == my_y
                )
                p_r = sel.astype(jnp.bfloat16)
                chunk = xslab[h * 128 : (h + 1) * 128, :]
                acc = acc + jnp.dot(
                    p_r, chunk, preferred_element_type=jnp.float32
                )
        out_ref[...] = acc

        for k in range(NXC):
            rdma_x[k].wait_send()

    return pl.pallas_call(
        body,
        out_shape=jax.ShapeDtypeStruct((T, D), jnp.float32),
        in_specs=[
            pl.BlockSpec(memory_space=pltpu.VMEM),
            pl.BlockSpec(memory_space=pltpu.VMEM),
        ],
        out_specs=pl.BlockSpec(memory_space=pltpu.VMEM),
        scratch_shapes=[
            pltpu.VMEM((2, T, D), jnp.bfloat16),
            pltpu.VMEM((2, RS, 128), jnp.int32),
            pltpu.SemaphoreType.DMA((2 + 2 * NXC,)),
        ],
        compiler_params=pltpu.CompilerParams(collective_id=0),
    )(x, d2)


def kernel(x, dest):
    return _a2a_kernel(x, dest.reshape(RS, 128))


# device time: 18928 ns/iter; 1.2349x vs baseline; 1.0134x over previous
import jax
import jax.numpy as jnp
from jax import lax
from jax.experimental import pallas as pl
from jax.experimental.pallas import tpu as pltpu

T = 1024
D = 512
RS = 8
NCH = 2 * T // 128
NXC = 4
XC = T // NXC
MC = XC // 128


def _a2a_kernel(x, d2):
    def body(x_ref, d_ref, out_ref, comm_x, comm_d, sems):
        my_x = lax.axis_index("x")
        my_y = lax.axis_index("y")
        my_z = lax.axis_index("z")
        peer = (my_x, 1 - my_y, my_z)

        comm_d[my_y] = d_ref[...]
        comm_x[my_y] = x_ref[...].astype(jnp.bfloat16)

        barrier = pltpu.get_barrier_semaphore()
        pl.semaphore_signal(
            barrier, inc=1, device_id=peer, device_id_type=pl.DeviceIdType.MESH
        )
        pl.semaphore_wait(barrier, 1)

        rdma_d = pltpu.make_async_remote_copy(
            src_ref=comm_d.at[my_y],
            dst_ref=comm_d.at[my_y],
            send_sem=sems.at[0],
            recv_sem=sems.at[1],
            device_id=peer,
            device_id_type=pl.DeviceIdType.MESH,
        )
        rdma_d.start()
        rdma_x = []
        for k in range(NXC):
            r = pltpu.make_async_remote_copy(
                src_ref=comm_x.at[my_y, pl.ds(k * XC, XC), :],
                dst_ref=comm_x.at[my_y, pl.ds(k * XC, XC), :],
                send_sem=sems.at[2 + 2 * k],
                recv_sem=sems.at[3 + 2 * k],
                device_id=peer,
                device_id_type=pl.DeviceIdType.MESH,
            )
            r.start()
            rdma_x.append(r)

        i0 = lax.broadcasted_iota(jnp.int32, (128, 128), 0)
        i1 = lax.broadcasted_iota(jnp.int32, (128, 128), 1)
        upper = (i0 <= i1).astype(jnp.float32)
        s0 = lax.broadcasted_iota(jnp.int32, (NCH, NCH), 0)
        s1 = lax.broadcasted_iota(jnp.int32, (NCH, NCH), 1)
        strict = (s1 < s0).astype(jnp.float32)
        jio = lax.broadcasted_iota(jnp.int32, (T, 128), 0)

        rdma_d.wait()

        dall = comm_d[...].reshape(NCH, 128)
        m = (dall == my_y).astype(jnp.float32)
        rowpre = jnp.dot(m, upper, preferred_element_type=jnp.float32)
        totals = rowpre[:, 127:128]
        rowoff = jnp.dot(strict, totals, preferred_element_type=jnp.float32)
        pos = (rowoff + rowpre - m).astype(jnp.int32)

        pos = pos + (dall != my_y).astype(jnp.int32) * (2 * T)

        im0 = my_y == 0
        pos_loc = jnp.where(im0, pos[:RS], pos[RS:])
        pos_rem = jnp.where(im0, pos[RS:], pos[:RS])

        def onehot(p8, rr):
            return (p8[rr : rr + 1, :] == jio).astype(jnp.bfloat16)

        p_loc = jnp.concatenate([onehot(pos_loc, rr) for rr in range(RS)], 1)
        acc = jnp.dot(p_loc, comm_x[my_y], preferred_element_type=jnp.float32)

        for k in range(NXC):
            rdma_x[k].wait_recv()
            xslab = comm_x[1 - my_y, pl.ds(k * XC, XC), :]
            p_k = jnp.concatenate(
                [onehot(pos_rem, k * MC + h) for h in range(MC)], 1
            )
            acc = acc + jnp.dot(p_k, xslab, preferred_element_type=jnp.float32)
        out_ref[...] = acc

        for k in range(NXC):
            rdma_x[k].wait_send()

    return pl.pallas_call(
        body,
        out_shape=jax.ShapeDtypeStruct((T, D), jnp.float32),
        in_specs=[
            pl.BlockSpec(memory_space=pltpu.VMEM),
            pl.BlockSpec(memory_space=pltpu.VMEM),
        ],
        out_specs=pl.BlockSpec(memory_space=pltpu.VMEM),
        scratch_shapes=[
            pltpu.VMEM((2, T, D), jnp.bfloat16),
            pltpu.VMEM((2, RS, 128), jnp.int32),
            pltpu.SemaphoreType.DMA((2 + 2 * NXC,)),
        ],
        compiler_params=pltpu.CompilerParams(collective_id=0),
    )(x, d2)


def kernel(x, dest):
    return _a2a_kernel(x, dest.reshape(RS, 128))
